# baseline (speedup 1.0000x reference)
import functools

import jax
import jax.numpy as jnp
from jax import lax
from jax.experimental import pallas as pl
from jax.experimental.pallas import tpu as pltpu
from jax.experimental.pallas import tpu_sc as plsc

_D = 128
_T = 128
_NC = 2
_NS = 16
_NW = _NC * _NS
_CH = 128


@functools.partial(jax.jit, static_argnums=0)
def _gather(B, idx2d, tbl):
    n_ch = B // (_NW * _CH)

    def body(idx_hbm, table_hbm, out_hbm, idx_v, table_v, rows_v, osem):
        wid = lax.axis_index("s") * _NC + lax.axis_index("c")
        row0 = wid * n_ch
        pltpu.sync_copy(table_hbm, table_v)
        pltpu.sync_copy(idx_hbm.at[pl.ds(row0, n_ch), :], idx_v)

        lanes = lax.iota(jnp.int32, 16)

        def out_slice(it):
            return out_hbm.at[pl.ds((row0 + it) * _CH * _D, _CH * _D)]

        blk = _CH * _D

        def pair(p, carry):
            for b in range(2):
                it = p * 2 + b

                @pl.when(p >= 1)
                def _():
                    pltpu.make_async_copy(
                        rows_v.at[pl.ds(b * blk, blk)], out_slice(it - 2), osem
                    ).wait()

                def group(g, sidx0, _it=it):
                    zv = idx_v[_it, pl.ds(g * 16, 16)]
                    gidx = zv * _D
                    sidx = sidx0
                    for _ in range(_D):
                        vals = plsc.load_gather(table_v, [gidx])
                        plsc.store_scatter(rows_v, [sidx], vals)
                        gidx = gidx + 1
                        sidx = sidx + 1
                    return sidx0 + 16 * _D

                lax.fori_loop(0, _CH // 16, group, lanes * _D + b * blk)
                pltpu.async_copy(
                    rows_v.at[pl.ds(b * blk, blk)], out_slice(it), osem
                )
            return carry

        lax.fori_loop(0, n_ch // 2, pair, 0)
        for k in (2, 1):
            it = n_ch - k
            pltpu.make_async_copy(
                rows_v.at[pl.ds((it % 2) * blk, blk)], out_slice(it), osem
            ).wait()

    mesh = plsc.VectorSubcoreMesh(core_axis_name="c", subcore_axis_name="s")
    f = pl.kernel(
        body,
        out_type=jax.ShapeDtypeStruct((B * _D,), jnp.float32),
        mesh=mesh,
        compiler_params=pltpu.CompilerParams(needs_layout_passes=False),
        scratch_types=[
            pltpu.VMEM((n_ch, _CH), jnp.int32),
            pltpu.VMEM((_T * _D,), jnp.float32),
            pltpu.VMEM((2 * _CH * _D,), jnp.float32),
            pltpu.SemaphoreType.DMA,
        ],
    )
    return f(idx2d, tbl)


def kernel(Z, table):
    n, m = Z.shape
    B = n * m
    idx2d = Z.reshape(B // _CH, _CH).astype(jnp.int32)
    tbl = table.at[0].set(0.0).reshape(-1)
    out = _gather(B, idx2d, tbl)
    return out.reshape(n, m, _D)


# speedup vs baseline: 18.4445x; 18.4445x over previous
import functools

import jax
import jax.numpy as jnp
from jax import lax
from jax.experimental import pallas as pl
from jax.experimental.pallas import tpu as pltpu
from jax.experimental.pallas import tpu_sc as plsc

_D = 128
_T = 128
_NC = 2
_NS = 16
_NW = _NC * _NS
_CH = 128


@functools.partial(jax.jit, static_argnums=0)
def _gather(B, idx2d, tbl):
    n_ch = B // (_NW * _CH)

    def body(idx_hbm, table_hbm, out_hbm, idx_v, table_v, rows_v, gsem):
        wid = lax.axis_index("s") * _NC + lax.axis_index("c")
        row0 = wid * n_ch

        @pl.when(lax.axis_index("s") == 0)
        def _():
            pltpu.sync_copy(table_hbm, table_v)

        pltpu.sync_copy(idx_hbm.at[pl.ds(row0, n_ch), :], idx_v)
        plsc.subcore_barrier()

        pltpu.async_copy(table_v.at[idx_v.at[0]], rows_v.at[0], gsem)

        def chunk(g, carry):
            b = lax.rem(g, 2)
            pltpu.make_async_copy(
                table_v.at[idx_v.at[g]], rows_v.at[b], gsem
            ).wait()
            gn = jnp.minimum(g + 1, n_ch - 1)
            pltpu.async_copy(table_v.at[idx_v.at[gn]], rows_v.at[1 - b], gsem)
            pltpu.sync_copy(rows_v.at[b], out_hbm.at[pl.ds((row0 + g) * _CH, _CH), :])
            return carry

        lax.fori_loop(0, n_ch, chunk, 0)
        pltpu.make_async_copy(
            table_v.at[idx_v.at[n_ch - 1]], rows_v.at[lax.rem(n_ch, 2)], gsem
        ).wait()

    mesh = plsc.VectorSubcoreMesh(core_axis_name="c", subcore_axis_name="s")
    f = pl.kernel(
        body,
        out_type=jax.ShapeDtypeStruct((B, _D), jnp.float32),
        mesh=mesh,
        scratch_types=[
            pltpu.VMEM((n_ch, _CH), jnp.int32),
            pltpu.VMEM_SHARED((_T, _D), jnp.float32),
            pltpu.VMEM((2, _CH, _D), jnp.float32),
            pltpu.SemaphoreType.DMA,
        ],
    )
    return f(idx2d, tbl)


def kernel(Z, table):
    n, m = Z.shape
    B = n * m
    idx2d = Z.reshape(B // _CH, _CH).astype(jnp.int32)
    tbl = table.at[0].set(0.0)
    out = _gather(B, idx2d, tbl)
    return out.reshape(n, m, _D)


# speedup vs baseline: 18.5598x; 1.0062x over previous
import functools

import jax
import jax.numpy as jnp
from jax import lax
from jax.experimental import pallas as pl
from jax.experimental.pallas import tpu as pltpu
from jax.experimental.pallas import tpu_sc as plsc

_D = 128
_T = 128
_NC = 2
_NS = 16
_NW = _NC * _NS
_CH = 128


@functools.partial(jax.jit, static_argnums=0)
def _gather(B, idx2d, tbl):
    n_ch = B // (_NW * _CH)

    def body(idx_hbm, table_hbm, out_hbm, idx_v, table_v, rows_v, zrow_v, gsem):
        wid = lax.axis_index("s") * _NC + lax.axis_index("c")
        row0 = wid * n_ch

        @pl.when(lax.axis_index("s") == 0)
        def _():
            pltpu.sync_copy(table_hbm, table_v)
            for j in range(_D // 16):
                zrow_v[0, pl.ds(j * 16, 16)] = jnp.zeros((16,), jnp.float32)
            pltpu.sync_copy(zrow_v, table_v.at[pl.ds(0, 1), :])

        pltpu.sync_copy(idx_hbm.at[pl.ds(row0, n_ch), :], idx_v)
        plsc.subcore_barrier()

        pltpu.async_copy(table_v.at[idx_v.at[0]], rows_v.at[0], gsem)

        def chunk(g, carry):
            b = lax.rem(g, 2)
            pltpu.make_async_copy(
                table_v.at[idx_v.at[g]], rows_v.at[b], gsem
            ).wait()
            gn = jnp.minimum(g + 1, n_ch - 1)
            pltpu.async_copy(table_v.at[idx_v.at[gn]], rows_v.at[1 - b], gsem)
            pltpu.sync_copy(rows_v.at[b], out_hbm.at[pl.ds((row0 + g) * _CH, _CH), :])
            return carry

        lax.fori_loop(0, n_ch, chunk, 0)
        pltpu.make_async_copy(
            table_v.at[idx_v.at[n_ch - 1]], rows_v.at[lax.rem(n_ch, 2)], gsem
        ).wait()

    mesh = plsc.VectorSubcoreMesh(core_axis_name="c", subcore_axis_name="s")
    f = pl.kernel(
        body,
        out_type=jax.ShapeDtypeStruct((B, _D), jnp.float32),
        mesh=mesh,
        scratch_types=[
            pltpu.VMEM((n_ch, _CH), jnp.int32),
            pltpu.VMEM_SHARED((_T, _D), jnp.float32),
            pltpu.VMEM((2, _CH, _D), jnp.float32),
            pltpu.VMEM((1, _D), jnp.float32),
            pltpu.SemaphoreType.DMA,
        ],
    )
    return f(idx2d, tbl)


def kernel(Z, table):
    n, m = Z.shape
    B = n * m
    idx2d = Z.reshape(B // _CH, _CH).astype(jnp.int32)
    out = _gather(B, idx2d, table)
    return out.reshape(n, m, _D)


# speedup vs baseline: 19.2909x; 1.0394x over previous
import functools

import jax
import jax.numpy as jnp
from jax import lax
from jax.experimental import pallas as pl
from jax.experimental.pallas import tpu as pltpu
from jax.experimental.pallas import tpu_sc as plsc

_D = 128
_T = 128
_NC = 2
_NS = 16
_NW = _NC * _NS
_CH = 128
_K = 2


@functools.partial(jax.jit, static_argnums=0)
def _gather(B, idx2d, tbl):
    n_ch = B // (_NW * _CH)

    def body(idx_hbm, table_hbm, out_hbm, idx_v, table_v, rows_v, zrow_v, gsem):
        wid = lax.axis_index("s") * _NC + lax.axis_index("c")
        row0 = wid * n_ch

        @pl.when(lax.axis_index("s") == 0)
        def _():
            pltpu.sync_copy(table_hbm, table_v)
            for j in range(_D // 16):
                zrow_v[0, pl.ds(j * 16, 16)] = jnp.zeros((16,), jnp.float32)
            pltpu.sync_copy(zrow_v, table_v.at[pl.ds(0, 1), :])

        pltpu.sync_copy(idx_hbm.at[pl.ds(row0, n_ch), :], idx_v)
        plsc.subcore_barrier()

        n_it = n_ch // _K

        def launch(it, b):
            for j in range(_K):
                pltpu.async_copy(
                    table_v.at[idx_v.at[it * _K + j]],
                    rows_v.at[b, pl.ds(j * _CH, _CH)],
                    gsem,
                )

        def drain(it, b):
            for j in range(_K):
                pltpu.make_async_copy(
                    table_v.at[idx_v.at[it * _K + j]],
                    rows_v.at[b, pl.ds(j * _CH, _CH)],
                    gsem,
                ).wait()

        launch(0, 0)

        def step(it, carry):
            b = lax.rem(it, 2)
            drain(it, b)
            launch(jnp.minimum(it + 1, n_it - 1), 1 - b)
            pltpu.sync_copy(
                rows_v.at[b],
                out_hbm.at[pl.ds((row0 + it * _K) * _CH, _K * _CH), :],
            )
            return carry

        lax.fori_loop(0, n_it, step, 0)
        drain(n_it - 1, lax.rem(n_it, 2))

    mesh = plsc.VectorSubcoreMesh(core_axis_name="c", subcore_axis_name="s")
    f = pl.kernel(
        body,
        out_type=jax.ShapeDtypeStruct((B, _D), jnp.float32),
        mesh=mesh,
        scratch_types=[
            pltpu.VMEM((n_ch, _CH), jnp.int32),
            pltpu.VMEM_SHARED((_T, _D), jnp.float32),
            pltpu.VMEM((2, _K * _CH, _D), jnp.float32),
            pltpu.VMEM((1, _D), jnp.float32),
            pltpu.SemaphoreType.DMA,
        ],
    )
    return f(idx2d, tbl)


def kernel(Z, table):
    n, m = Z.shape
    B = n * m
    idx2d = Z.reshape(B // _CH, _CH).astype(jnp.int32)
    out = _gather(B, idx2d, table)
    return out.reshape(n, m, _D)
